# initial kernel scaffold (unmeasured)
import jax
import jax.numpy as jnp
from jax import lax
from jax.experimental import pallas as pl
from jax.experimental.pallas import tpu as pltpu


def kernel(
    x,
):
    def body(*refs):
        pass

    out_shape = jax.ShapeDtypeStruct(..., jnp.float32)
    return pl.pallas_call(body, out_shape=out_shape)(...)


# baseline (device time: 15538 ns/iter reference)
import jax
import jax.numpy as jnp
from jax import lax
from jax.experimental import pallas as pl
from jax.experimental.pallas import tpu as pltpu

N_DEV = 16


def kernel(x):
    m_per, n = x.shape

    def body(x_ref, out_ref, comm_ref, send_sems, recv_sems):
        my_pos = lax.axis_index("i")

        comm_ref[pl.ds(my_pos, 1), :] = jnp.sum(
            x_ref[...], axis=0, keepdims=True
        )

        barrier_sem = pltpu.get_barrier_semaphore()
        for d in range(1, N_DEV):
            peer = lax.rem(my_pos + d, N_DEV)
            pl.semaphore_signal(
                barrier_sem,
                inc=1,
                device_id=(peer,),
                device_id_type=pl.DeviceIdType.MESH,
            )
        pl.semaphore_wait(barrier_sem, N_DEV - 1)

        for d in range(1, N_DEV):
            peer = lax.rem(my_pos + d, N_DEV)
            rdma = pltpu.make_async_remote_copy(
                src_ref=comm_ref.at[pl.ds(my_pos, 1), :],
                dst_ref=comm_ref.at[pl.ds(my_pos, 1), :],
                send_sem=send_sems.at[d],
                recv_sem=recv_sems.at[d],
                device_id=(peer,),
                device_id_type=pl.DeviceIdType.MESH,
            )
            rdma.start()

        for d in range(1, N_DEV):
            src_pos = lax.rem(my_pos - d + N_DEV, N_DEV)
            recv = pltpu.make_async_remote_copy(
                src_ref=comm_ref.at[pl.ds(my_pos, 1), :],
                dst_ref=comm_ref.at[pl.ds(src_pos, 1), :],
                send_sem=send_sems.at[d],
                recv_sem=recv_sems.at[d],
                device_id=(src_pos,),
                device_id_type=pl.DeviceIdType.MESH,
            )
            recv.wait_recv()

        out_ref[...] = jnp.sum(comm_ref[...], axis=0, keepdims=True)

        for d in range(1, N_DEV):
            send = pltpu.make_async_remote_copy(
                src_ref=comm_ref.at[pl.ds(my_pos, 1), :],
                dst_ref=comm_ref.at[pl.ds(my_pos, 1), :],
                send_sem=send_sems.at[d],
                recv_sem=recv_sems.at[d],
                device_id=(0,),
                device_id_type=pl.DeviceIdType.MESH,
            )
            send.wait_send()

    return pl.pallas_call(
        body,
        out_shape=jax.ShapeDtypeStruct((1, n), jnp.float32),
        in_specs=[pl.BlockSpec(memory_space=pltpu.VMEM)],
        out_specs=pl.BlockSpec(memory_space=pltpu.VMEM),
        scratch_shapes=[
            pltpu.VMEM((N_DEV, n), jnp.float32),
            pltpu.SemaphoreType.DMA((N_DEV,)),
            pltpu.SemaphoreType.DMA((N_DEV,)),
        ],
        compiler_params=pltpu.CompilerParams(collective_id=0),
    )(x)


# device time: 15012 ns/iter; 1.0350x vs baseline; 1.0350x over previous
import jax
import jax.numpy as jnp
from jax import lax
from jax.experimental import pallas as pl
from jax.experimental.pallas import tpu as pltpu

N_DEV = 16
BLOCK_M = 512


def kernel(x):
    m_per, n = x.shape
    n_steps = m_per // BLOCK_M

    def body(x_ref, out_ref, acc_ref, comm_ref, send_sems, recv_sems):
        my_pos = lax.axis_index("i")
        i = pl.program_id(0)

        @pl.when(i == 0)
        def _():
            barrier_sem = pltpu.get_barrier_semaphore()
            for d in range(1, N_DEV):
                peer = lax.rem(my_pos + d, N_DEV)
                pl.semaphore_signal(
                    barrier_sem,
                    inc=1,
                    device_id=(peer,),
                    device_id_type=pl.DeviceIdType.MESH,
                )
            acc_ref[...] = jnp.sum(x_ref[...], axis=0, keepdims=True)

        @pl.when(i > 0)
        def _():
            acc_ref[...] += jnp.sum(x_ref[...], axis=0, keepdims=True)

        @pl.when(i == n_steps - 1)
        def _():
            comm_ref[pl.ds(my_pos, 1), :] = acc_ref[...]
            barrier_sem = pltpu.get_barrier_semaphore()
            pl.semaphore_wait(barrier_sem, N_DEV - 1)

            for d in range(1, N_DEV):
                peer = lax.rem(my_pos + d, N_DEV)
                rdma = pltpu.make_async_remote_copy(
                    src_ref=comm_ref.at[pl.ds(my_pos, 1), :],
                    dst_ref=comm_ref.at[pl.ds(my_pos, 1), :],
                    send_sem=send_sems.at[d],
                    recv_sem=recv_sems.at[d],
                    device_id=(peer,),
                    device_id_type=pl.DeviceIdType.MESH,
                )
                rdma.start()

            for d in range(1, N_DEV):
                src_pos = lax.rem(my_pos - d + N_DEV, N_DEV)
                recv = pltpu.make_async_remote_copy(
                    src_ref=comm_ref.at[pl.ds(my_pos, 1), :],
                    dst_ref=comm_ref.at[pl.ds(src_pos, 1), :],
                    send_sem=send_sems.at[d],
                    recv_sem=recv_sems.at[d],
                    device_id=(src_pos,),
                    device_id_type=pl.DeviceIdType.MESH,
                )
                recv.wait_recv()

            out_ref[...] = jnp.sum(comm_ref[...], axis=0, keepdims=True)

            for d in range(1, N_DEV):
                send = pltpu.make_async_remote_copy(
                    src_ref=comm_ref.at[pl.ds(my_pos, 1), :],
                    dst_ref=comm_ref.at[pl.ds(my_pos, 1), :],
                    send_sem=send_sems.at[d],
                    recv_sem=recv_sems.at[d],
                    device_id=(0,),
                    device_id_type=pl.DeviceIdType.MESH,
                )
                send.wait_send()

    return pl.pallas_call(
        body,
        grid=(n_steps,),
        out_shape=jax.ShapeDtypeStruct((1, n), jnp.float32),
        in_specs=[
            pl.BlockSpec((BLOCK_M, n), lambda i: (i, 0), memory_space=pltpu.VMEM)
        ],
        out_specs=pl.BlockSpec((1, n), lambda i: (0, 0), memory_space=pltpu.VMEM),
        scratch_shapes=[
            pltpu.VMEM((1, n), jnp.float32),
            pltpu.VMEM((N_DEV, n), jnp.float32),
            pltpu.SemaphoreType.DMA((N_DEV,)),
            pltpu.SemaphoreType.DMA((N_DEV,)),
        ],
        compiler_params=pltpu.CompilerParams(collective_id=0),
    )(x)
